# Initial kernel scaffold; baseline (speedup 1.0000x reference)
import jax
import jax.numpy as jnp
from jax.experimental import pallas as pl


def kernel(x, edge_index, mention_index, W1l, b1l, W1r, W2l, b2l, W2r):
    raise NotImplementedError("write your pallas kernel here")


# speedup vs baseline: 4.6679x; 4.6679x over previous
import jax
import jax.numpy as jnp
from jax import lax
from jax.experimental import pallas as pl
from jax.experimental.pallas import tpu as pltpu
from jax.experimental.pallas import tpu_sc as plsc

N = 10000
E = 320000
D = 128

NC = 2
NS = 16
NW = NC * NS
EPW = E // NW
EK = 100
ENCH = EPW // EK

ZROWS = 624
ZTAIL0 = NS * ZROWS
ZTAIL = N - ZTAIL0


def _sc_aggregate(table, src2, dst2, zD):
    mesh = plsc.VectorSubcoreMesh(core_axis_name="c", subcore_axis_name="s")

    def body(table_r, src_r, dst_r, zD_r, pacc_r,
             acc_sh, src_c, dst_c, rows_v, gsem):
        c = lax.axis_index("c")
        s = lax.axis_index("s")
        wid = c * NS + s

        z0 = s * ZROWS
        pltpu.sync_copy(zD_r.at[pl.ds(z0, ZROWS)], acc_sh.at[pl.ds(z0, ZROWS)])

        @pl.when(s == 0)
        def _zero_tail():
            pltpu.sync_copy(zD_r.at[pl.ds(ZTAIL0, ZTAIL)],
                            acc_sh.at[pl.ds(ZTAIL0, ZTAIL)])

        plsc.subcore_barrier()

        def chunk(j, carry):
            pltpu.sync_copy(src_r.at[wid, j], src_c)
            pltpu.sync_copy(dst_r.at[wid, j], dst_c)
            pltpu.async_copy(table_r.at[src_c], rows_v, gsem).wait()
            pltpu.sync_copy(rows_v, acc_sh.at[dst_c], add=True)
            return carry

        lax.fori_loop(0, ENCH, chunk, 0)
        plsc.subcore_barrier()

        pltpu.sync_copy(acc_sh.at[pl.ds(z0, ZROWS)],
                        pacc_r.at[c, pl.ds(z0, ZROWS)])

        @pl.when(s == 0)
        def _out_tail():
            pltpu.sync_copy(acc_sh.at[pl.ds(ZTAIL0, ZTAIL)],
                            pacc_r.at[c, pl.ds(ZTAIL0, ZTAIL)])

    return pl.kernel(
        body,
        out_type=jax.ShapeDtypeStruct((NC, N, D), jnp.float32),
        mesh=mesh,
        scratch_types=[
            pltpu.VMEM_SHARED((N, D), jnp.float32),
            pltpu.VMEM((EK,), jnp.int32),
            pltpu.VMEM((EK,), jnp.int32),
            pltpu.VMEM((EK, D), jnp.float32),
            pltpu.SemaphoreType.DMA,
        ],
    )(table, src2, dst2, zD)


def _sc_counts(dst2, zD):
    mesh = plsc.VectorSubcoreMesh(core_axis_name="c", subcore_axis_name="s")

    def body(dst_r, zD_r, pcnt_r, cnt_sh, dst_c, ones_v, gsem):
        c = lax.axis_index("c")
        s = lax.axis_index("s")
        wid = c * NS + s

        z0 = s * ZROWS
        pltpu.sync_copy(zD_r.at[pl.ds(z0, ZROWS)], cnt_sh.at[pl.ds(z0, ZROWS)])

        @pl.when(s == 0)
        def _zero_tail():
            pltpu.sync_copy(zD_r.at[pl.ds(ZTAIL0, ZTAIL)],
                            cnt_sh.at[pl.ds(ZTAIL0, ZTAIL)])

        def ones_row(i, carry):
            for k in range(D // 16):
                ones_v[i, pl.ds(k * 16, 16)] = jnp.ones((16,), jnp.float32)
            return carry

        lax.fori_loop(0, EK, ones_row, 0)
        plsc.subcore_barrier()

        def chunk(j, carry):
            pltpu.sync_copy(dst_r.at[wid, j], dst_c)
            pltpu.sync_copy(ones_v, cnt_sh.at[dst_c], add=True)
            return carry

        lax.fori_loop(0, ENCH, chunk, 0)
        plsc.subcore_barrier()

        pltpu.sync_copy(cnt_sh.at[pl.ds(z0, ZROWS)],
                        pcnt_r.at[c, pl.ds(z0, ZROWS)])

        @pl.when(s == 0)
        def _out_tail():
            pltpu.sync_copy(cnt_sh.at[pl.ds(ZTAIL0, ZTAIL)],
                            pcnt_r.at[c, pl.ds(ZTAIL0, ZTAIL)])

    return pl.kernel(
        body,
        out_type=jax.ShapeDtypeStruct((NC, N, D), jnp.float32),
        mesh=mesh,
        scratch_types=[
            pltpu.VMEM_SHARED((N, D), jnp.float32),
            pltpu.VMEM((EK,), jnp.int32),
            pltpu.VMEM((EK, D), jnp.float32),
            pltpu.SemaphoreType.DMA,
        ],
    )(dst2, zD)


def _tc_dense1_body(pacc, pcnt, x, wl, bl, wr, h_out, inv_out):
    cnt = pcnt[0, :, 0:1] + pcnt[1, :, 0:1]
    inv = 1.0 / jnp.maximum(cnt, 1.0)
    agg = (pacc[0] + pacc[1]) * inv
    pre = (
        lax.dot_general(agg, wl[...], (((1,), (1,)), ((), ())),
                        preferred_element_type=jnp.float32)
        + bl[...]
        + lax.dot_general(x[...], wr[...], (((1,), (1,)), ((), ())),
                          preferred_element_type=jnp.float32)
    )
    h_out[...] = jnp.maximum(pre, 0.0)
    inv_out[...] = inv


def _tc_dense2_body(pacc, inv, h, wl, bl, wr, h2_out):
    agg = (pacc[0] + pacc[1]) * inv[...]
    h2_out[...] = (
        lax.dot_general(agg, wl[...], (((1,), (1,)), ((), ())),
                        preferred_element_type=jnp.float32)
        + bl[...]
        + lax.dot_general(h[...], wr[...], (((1,), (1,)), ((), ())),
                          preferred_element_type=jnp.float32)
    )


RPW = 312
RK = 104
RNCH = RPW // RK
RTAIL = N - NW * RPW


def _sc_scores(h2, mention):
    mesh = plsc.VectorSubcoreMesh(core_axis_name="c", subcore_axis_name="s")

    def body(h2_r, m_r, sc_r, midx_v, grows_v, hrows_v, scores_v, gsem):
        c = lax.axis_index("c")
        s = lax.axis_index("s")
        wid = c * NS + s
        base = wid * RPW

        def row_dot(g_ref, h_ref, r):
            acc = jnp.zeros((16,), jnp.float32)
            for k in range(D // 16):
                acc = acc + g_ref[r, pl.ds(k * 16, 16)] * h_ref[r, pl.ds(k * 16, 16)]
            return acc

        def do_chunk(off, nrows):
            pltpu.sync_copy(m_r.at[pl.ds(off, nrows)],
                            midx_v.at[pl.ds(0, nrows)])
            pltpu.async_copy(h2_r.at[midx_v.at[pl.ds(0, nrows)]],
                             grows_v.at[pl.ds(0, nrows)], gsem).wait()
            pltpu.sync_copy(h2_r.at[pl.ds(off, nrows)],
                            hrows_v.at[pl.ds(0, nrows)])

            def row(r, carry2):
                scores_v[r] = row_dot(grows_v, hrows_v, r)
                return carry2

            lax.fori_loop(0, nrows, row, 0)
            pltpu.sync_copy(scores_v.at[pl.ds(0, nrows)],
                            sc_r.at[pl.ds(off, nrows)])

        def chunk(j, carry):
            do_chunk(base + j * RK, RK)
            return carry

        lax.fori_loop(0, RNCH, chunk, 0)

        @pl.when(wid == 0)
        def _tail():
            do_chunk(NW * RPW, RTAIL)

    return pl.kernel(
        body,
        out_type=jax.ShapeDtypeStruct((N, 16), jnp.float32),
        mesh=mesh,
        scratch_types=[
            pltpu.VMEM((RK,), jnp.int32),
            pltpu.VMEM((RK, D), jnp.float32),
            pltpu.VMEM((RK, D), jnp.float32),
            pltpu.VMEM((RK, 16), jnp.float32),
            pltpu.SemaphoreType.DMA,
        ],
    )(h2, mention)


def _tc_softmax_body(s_ref, z_ref):
    s = jnp.sum(s_ref[...], axis=1)
    m = jnp.max(s)
    e = jnp.exp(s - m)
    z_ref[...] = e / jnp.sum(e)


def kernel(x, edge_index, mention_index, W1l, b1l, W1r, W2l, b2l, W2r):
    src2 = edge_index[0].astype(jnp.int32).reshape(NW, ENCH, EK)
    dst2 = edge_index[1].astype(jnp.int32).reshape(NW, ENCH, EK)
    mention = mention_index.astype(jnp.int32)
    zD = jnp.zeros((N, D), jnp.float32)
    bl1 = b1l.reshape(1, D)
    bl2 = b2l.reshape(1, D)

    pcnt = _sc_counts(dst2, zD)

    pacc1 = _sc_aggregate(x, src2, dst2, zD)
    h, inv = pl.pallas_call(
        _tc_dense1_body,
        out_shape=[
            jax.ShapeDtypeStruct((N, D), jnp.float32),
            jax.ShapeDtypeStruct((N, 1), jnp.float32),
        ],
    )(pacc1, pcnt, x, W1l, bl1, W1r)

    pacc2 = _sc_aggregate(h, src2, dst2, zD)
    h2 = pl.pallas_call(
        _tc_dense2_body,
        out_shape=jax.ShapeDtypeStruct((N, D), jnp.float32),
    )(pacc2, inv, h, W2l, bl2, W2r)

    scores16 = _sc_scores(h2, mention)
    z = pl.pallas_call(
        _tc_softmax_body,
        out_shape=jax.ShapeDtypeStruct((N,), jnp.float32),
    )(scores16)
    return z


# speedup vs baseline: 8.4929x; 1.8194x over previous
import jax
import jax.numpy as jnp
from jax import lax
from jax.experimental import pallas as pl
from jax.experimental.pallas import tpu as pltpu
from jax.experimental.pallas import tpu_sc as plsc

N = 10000
E = 320000
D = 128

NC = 2
NS = 16
NW = NC * NS
EPW = E // NW
EK = 125
ENCH = EPW // EK
ENCH2 = ENCH // 2

ZROWS = 624
ZTAIL0 = NS * ZROWS
ZTAIL = N - ZTAIL0


def _sc_aggregate(table, src2, dst2, zD):
    mesh = plsc.VectorSubcoreMesh(core_axis_name="c", subcore_axis_name="s")

    def body(table_r, src_r, dst_r, zD_r, pacc_r, acc_sh,
             src_a, dst_a, rows_a, isem_a, gsem_a,
             src_b, dst_b, rows_b, isem_b, gsem_b):
        c = lax.axis_index("c")
        s = lax.axis_index("s")
        wid = c * NS + s

        z0 = s * ZROWS
        pltpu.sync_copy(zD_r.at[pl.ds(z0, ZROWS)], acc_sh.at[pl.ds(z0, ZROWS)])

        @pl.when(s == 0)
        def _zero_tail():
            pltpu.sync_copy(zD_r.at[pl.ds(ZTAIL0, ZTAIL)],
                            acc_sh.at[pl.ds(ZTAIL0, ZTAIL)])

        def start_idx(j, sbuf, dbuf, sem):
            pltpu.async_copy(src_r.at[wid, j], sbuf, sem)
            pltpu.async_copy(dst_r.at[wid, j], dbuf, sem)

        def wait_idx(sbuf, dbuf, sem):
            pltpu.make_async_copy(src_r.at[wid, 0], sbuf, sem).wait()
            pltpu.make_async_copy(dst_r.at[wid, 0], dbuf, sem).wait()

        plsc.subcore_barrier()

        start_idx(0, src_a, dst_a, isem_a)
        wait_idx(src_a, dst_a, isem_a)
        pltpu.async_copy(table_r.at[src_a], rows_a, gsem_a)
        start_idx(1, src_b, dst_b, isem_b)

        def body2(jj, carry):
            not_last = jj < ENCH2 - 1
            pltpu.make_async_copy(table_r.at[src_a], rows_a, gsem_a).wait()
            wait_idx(src_b, dst_b, isem_b)
            pltpu.async_copy(table_r.at[src_b], rows_b, gsem_b)
            pltpu.sync_copy(rows_a, acc_sh.at[dst_a], add=True)

            @pl.when(not_last)
            def _pf_a():
                start_idx(2 * jj + 2, src_a, dst_a, isem_a)

            pltpu.make_async_copy(table_r.at[src_b], rows_b, gsem_b).wait()

            @pl.when(not_last)
            def _g_a():
                wait_idx(src_a, dst_a, isem_a)
                pltpu.async_copy(table_r.at[src_a], rows_a, gsem_a)

            pltpu.sync_copy(rows_b, acc_sh.at[dst_b], add=True)

            @pl.when(not_last)
            def _pf_b():
                start_idx(2 * jj + 3, src_b, dst_b, isem_b)

            return carry

        lax.fori_loop(0, ENCH2, body2, 0)
        plsc.subcore_barrier()

        pltpu.sync_copy(acc_sh.at[pl.ds(z0, ZROWS)],
                        pacc_r.at[c, pl.ds(z0, ZROWS)])

        @pl.when(s == 0)
        def _out_tail():
            pltpu.sync_copy(acc_sh.at[pl.ds(ZTAIL0, ZTAIL)],
                            pacc_r.at[c, pl.ds(ZTAIL0, ZTAIL)])

    return pl.kernel(
        body,
        out_type=jax.ShapeDtypeStruct((NC, N, D), jnp.float32),
        mesh=mesh,
        scratch_types=[
            pltpu.VMEM_SHARED((N, D), jnp.float32),
            pltpu.VMEM((EK,), jnp.int32),
            pltpu.VMEM((EK,), jnp.int32),
            pltpu.VMEM((EK, D), jnp.float32),
            pltpu.SemaphoreType.DMA,
            pltpu.SemaphoreType.DMA,
            pltpu.VMEM((EK,), jnp.int32),
            pltpu.VMEM((EK,), jnp.int32),
            pltpu.VMEM((EK, D), jnp.float32),
            pltpu.SemaphoreType.DMA,
            pltpu.SemaphoreType.DMA,
        ],
    )(table, src2, dst2, zD)


def _sc_counts(dst2, zD):
    mesh = plsc.VectorSubcoreMesh(core_axis_name="c", subcore_axis_name="s")

    def body(dst_r, zD_r, pcnt_r, cnt_sh, dst_a, dst_b, ones_v,
             isem_a, isem_b):
        c = lax.axis_index("c")
        s = lax.axis_index("s")
        wid = c * NS + s

        z0 = s * ZROWS
        pltpu.sync_copy(zD_r.at[pl.ds(z0, ZROWS)], cnt_sh.at[pl.ds(z0, ZROWS)])

        @pl.when(s == 0)
        def _zero_tail():
            pltpu.sync_copy(zD_r.at[pl.ds(ZTAIL0, ZTAIL)],
                            cnt_sh.at[pl.ds(ZTAIL0, ZTAIL)])

        def ones_row(i, carry):
            for k in range(D // 16):
                ones_v[i, pl.ds(k * 16, 16)] = jnp.ones((16,), jnp.float32)
            return carry

        lax.fori_loop(0, EK, ones_row, 0)
        plsc.subcore_barrier()

        pltpu.async_copy(dst_r.at[wid, 0], dst_a, isem_a)

        def body2(jj, carry):
            not_last = jj < ENCH2 - 1
            pltpu.make_async_copy(dst_r.at[wid, 0], dst_a, isem_a).wait()
            pltpu.async_copy(dst_r.at[wid, 2 * jj + 1], dst_b, isem_b)
            pltpu.sync_copy(ones_v, cnt_sh.at[dst_a], add=True)
            pltpu.make_async_copy(dst_r.at[wid, 0], dst_b, isem_b).wait()

            @pl.when(not_last)
            def _pf():
                pltpu.async_copy(dst_r.at[wid, 2 * jj + 2], dst_a, isem_a)

            pltpu.sync_copy(ones_v, cnt_sh.at[dst_b], add=True)
            return carry

        lax.fori_loop(0, ENCH2, body2, 0)
        plsc.subcore_barrier()

        pltpu.sync_copy(cnt_sh.at[pl.ds(z0, ZROWS)],
                        pcnt_r.at[c, pl.ds(z0, ZROWS)])

        @pl.when(s == 0)
        def _out_tail():
            pltpu.sync_copy(cnt_sh.at[pl.ds(ZTAIL0, ZTAIL)],
                            pcnt_r.at[c, pl.ds(ZTAIL0, ZTAIL)])

    return pl.kernel(
        body,
        out_type=jax.ShapeDtypeStruct((NC, N, D), jnp.float32),
        mesh=mesh,
        scratch_types=[
            pltpu.VMEM_SHARED((N, D), jnp.float32),
            pltpu.VMEM((EK,), jnp.int32),
            pltpu.VMEM((EK,), jnp.int32),
            pltpu.VMEM((EK, D), jnp.float32),
            pltpu.SemaphoreType.DMA,
            pltpu.SemaphoreType.DMA,
        ],
    )(dst2, zD)


def _tc_dense1_body(pacc, pcnt, x, wl, bl, wr, h_out, inv_out):
    cnt = pcnt[0, :, 0:1] + pcnt[1, :, 0:1]
    inv = 1.0 / jnp.maximum(cnt, 1.0)
    agg = (pacc[0] + pacc[1]) * inv
    pre = (
        lax.dot_general(agg, wl[...], (((1,), (1,)), ((), ())),
                        preferred_element_type=jnp.float32)
        + bl[...]
        + lax.dot_general(x[...], wr[...], (((1,), (1,)), ((), ())),
                          preferred_element_type=jnp.float32)
    )
    h_out[...] = jnp.maximum(pre, 0.0)
    inv_out[...] = inv


def _tc_dense2_body(pacc, inv, h, wl, bl, wr, h2_out):
    agg = (pacc[0] + pacc[1]) * inv[...]
    h2_out[...] = (
        lax.dot_general(agg, wl[...], (((1,), (1,)), ((), ())),
                        preferred_element_type=jnp.float32)
        + bl[...]
        + lax.dot_general(h[...], wr[...], (((1,), (1,)), ((), ())),
                          preferred_element_type=jnp.float32)
    )


RPW = 312
RK = 104
RNCH = RPW // RK
RTAIL = N - NW * RPW


def _sc_scores(h2, mention):
    mesh = plsc.VectorSubcoreMesh(core_axis_name="c", subcore_axis_name="s")

    def body(h2_r, m_r, sc_r, midx_v, grows_v, hrows_v, scores_v, gsem):
        c = lax.axis_index("c")
        s = lax.axis_index("s")
        wid = c * NS + s
        base = wid * RPW

        def row_dot(g_ref, h_ref, r):
            acc = jnp.zeros((16,), jnp.float32)
            for k in range(D // 16):
                acc = acc + g_ref[r, pl.ds(k * 16, 16)] * h_ref[r, pl.ds(k * 16, 16)]
            return acc

        def do_chunk(off, nrows):
            pltpu.sync_copy(m_r.at[pl.ds(off, nrows)],
                            midx_v.at[pl.ds(0, nrows)])
            pltpu.async_copy(h2_r.at[midx_v.at[pl.ds(0, nrows)]],
                             grows_v.at[pl.ds(0, nrows)], gsem).wait()
            pltpu.sync_copy(h2_r.at[pl.ds(off, nrows)],
                            hrows_v.at[pl.ds(0, nrows)])

            def row(r, carry2):
                scores_v[r] = row_dot(grows_v, hrows_v, r)
                return carry2

            lax.fori_loop(0, nrows, row, 0)
            pltpu.sync_copy(scores_v.at[pl.ds(0, nrows)],
                            sc_r.at[pl.ds(off, nrows)])

        def chunk(j, carry):
            do_chunk(base + j * RK, RK)
            return carry

        lax.fori_loop(0, RNCH, chunk, 0)

        @pl.when(wid == 0)
        def _tail():
            do_chunk(NW * RPW, RTAIL)

    return pl.kernel(
        body,
        out_type=jax.ShapeDtypeStruct((N, 16), jnp.float32),
        mesh=mesh,
        scratch_types=[
            pltpu.VMEM((RK,), jnp.int32),
            pltpu.VMEM((RK, D), jnp.float32),
            pltpu.VMEM((RK, D), jnp.float32),
            pltpu.VMEM((RK, 16), jnp.float32),
            pltpu.SemaphoreType.DMA,
        ],
    )(h2, mention)


def _tc_softmax_body(s_ref, z_ref):
    s = jnp.sum(s_ref[...], axis=1)
    m = jnp.max(s)
    e = jnp.exp(s - m)
    z_ref[...] = e / jnp.sum(e)


def kernel(x, edge_index, mention_index, W1l, b1l, W1r, W2l, b2l, W2r):
    src2 = edge_index[0].astype(jnp.int32).reshape(NW, ENCH, EK)
    dst2 = edge_index[1].astype(jnp.int32).reshape(NW, ENCH, EK)
    mention = mention_index.astype(jnp.int32)
    zD = jnp.zeros((N, D), jnp.float32)
    bl1 = b1l.reshape(1, D)
    bl2 = b2l.reshape(1, D)

    pcnt = _sc_counts(dst2, zD)

    pacc1 = _sc_aggregate(x, src2, dst2, zD)
    h, inv = pl.pallas_call(
        _tc_dense1_body,
        out_shape=[
            jax.ShapeDtypeStruct((N, D), jnp.float32),
            jax.ShapeDtypeStruct((N, 1), jnp.float32),
        ],
    )(pacc1, pcnt, x, W1l, bl1, W1r)

    pacc2 = _sc_aggregate(h, src2, dst2, zD)
    h2 = pl.pallas_call(
        _tc_dense2_body,
        out_shape=jax.ShapeDtypeStruct((N, D), jnp.float32),
    )(pacc2, inv, h, W2l, bl2, W2r)

    scores16 = _sc_scores(h2, mention)
    z = pl.pallas_call(
        _tc_softmax_body,
        out_shape=jax.ShapeDtypeStruct((N,), jnp.float32),
    )(scores16)
    return z
